# baseline (speedup 1.0000x reference)
import functools
import math

import jax
import jax.numpy as jnp
from jax import lax
from jax.experimental import pallas as pl
from jax.experimental.pallas import tpu as pltpu


def _fff_block_kernel(x_ref, w1hs_ref, w1ls_ref, w1hd_ref, w1ld_ref, w2_ref,
                      out_ref, *, depth, n_pad, n_shallow, shallow_levels):
    x = x_ref[...]
    m = x.shape[0]
    bits = lax.bitcast_convert_type(x, jnp.uint32)
    rounded = (bits + jnp.uint32(0x7FFF) + ((bits >> 16) & jnp.uint32(1))) \
        & jnp.uint32(0xFFFF0000)
    hi = lax.bitcast_convert_type(rounded, jnp.float32)
    xh = hi.astype(jnp.bfloat16)
    xl = (x - hi).astype(jnp.bfloat16)

    dn = (((1,), (1,)), ((), ()))
    f32 = jnp.float32
    w1hs = w1hs_ref[...]
    w1ls = w1ls_ref[...]
    ls_s = lax.dot_general(xh, w1hs, dn, preferred_element_type=f32)
    ls_s += lax.dot_general(xl, w1hs, dn, preferred_element_type=f32)
    ls_s += lax.dot_general(xh, w1ls, dn, preferred_element_type=f32)
    ls_s += lax.dot_general(xl, w1ls, dn, preferred_element_type=f32)

    w1hd = w1hd_ref[...]
    ls_d = lax.dot_general(xh, w1hd, dn, preferred_element_type=f32)
    ls_d += lax.dot_general(xl, w1hd, dn, preferred_element_type=f32)
    ls_d += lax.dot_general(xh, w1ld_ref[...], dn, preferred_element_type=f32)

    p = jnp.zeros((m, 1), jnp.int32)
    pieces = []
    for lvl in range(depth):
        w = 1 << lvl
        base = w - 1
        if lvl < shallow_levels:
            src = ls_s
            off = base
        else:
            src = ls_d
            off = base - n_shallow
        sl = lax.slice(src, (0, off), (m, off + w))
        col = lax.broadcasted_iota(jnp.int32, (m, w), 1)
        sel = col == p
        logit = jnp.sum(jnp.where(sel, sl, 0.0), axis=1, keepdims=True)
        act = jax.nn.gelu(logit)
        pieces.append(jnp.where(sel, act, 0.0))
        p = 2 * p + (logit > 0.0).astype(jnp.int32)
    n_nodes = (1 << depth) - 1
    if n_pad > n_nodes:
        pieces.append(jnp.zeros((m, n_pad - n_nodes), jnp.float32))
    acts = jnp.concatenate(pieces, axis=1).astype(jnp.bfloat16)

    out_ref[...] = lax.dot_general(
        acts, w2_ref[...], (((1,), (0,)), ((), ())),
        preferred_element_type=f32,
    )


def _split_bf16(a):
    bits = lax.bitcast_convert_type(a, jnp.uint32)
    rounded = (bits + jnp.uint32(0x7FFF) + ((bits >> 16) & jnp.uint32(1))) \
        & jnp.uint32(0xFFFF0000)
    hi = lax.bitcast_convert_type(rounded, jnp.float32)
    lo = a - hi
    return hi.astype(jnp.bfloat16), lo.astype(jnp.bfloat16)


@jax.jit
def kernel(input, w1s, w2s):
    tokens, nin = input.shape
    n_nodes, nout = w2s.shape
    depth = int(math.log2(n_nodes + 1))
    n_pad = n_nodes + 1

    shallow_levels = 8
    n_shallow = (1 << shallow_levels) - 1
    ns_pad = n_shallow + 1
    n_deep = n_nodes - n_shallow

    w1_s = jnp.concatenate(
        [w1s[:n_shallow], jnp.zeros((ns_pad - n_shallow, nin), w1s.dtype)])
    w1_d = w1s[n_shallow:]
    w1hs, w1ls = _split_bf16(w1_s)
    w1hd, w1ld = _split_bf16(w1_d)

    w2p = jnp.concatenate([w2s, jnp.zeros((n_pad - n_nodes, nout), w2s.dtype)])
    w2p = w2p.astype(jnp.bfloat16)

    m = 256
    grid = (tokens // m,)
    return pl.pallas_call(
        functools.partial(_fff_block_kernel, depth=depth, n_pad=n_pad,
                          n_shallow=n_shallow, shallow_levels=shallow_levels),
        grid=grid,
        in_specs=[
            pl.BlockSpec((m, nin), lambda i: (i, 0)),
            pl.BlockSpec((ns_pad, nin), lambda i: (0, 0)),
            pl.BlockSpec((ns_pad, nin), lambda i: (0, 0)),
            pl.BlockSpec((n_deep, nin), lambda i: (0, 0)),
            pl.BlockSpec((n_deep, nin), lambda i: (0, 0)),
            pl.BlockSpec((n_pad, nout), lambda i: (0, 0)),
        ],
        out_specs=pl.BlockSpec((m, nout), lambda i: (i, 0)),
        out_shape=jax.ShapeDtypeStruct((tokens, nout), jnp.float32),
    )(input, w1hs, w1ls, w1hd, w1ld, w2p)
